# baseline (speedup 1.0000x reference)
import functools

import jax
import jax.numpy as jnp
from jax import lax
from jax.experimental import pallas as pl
from jax.experimental.pallas import tpu as pltpu
from jax.experimental.pallas import tpu_sc as plsc

_B = 2048
_D = 768
_E = 8
_H = 1024
_K = 2
_TILE = 128
_NT = 40
_XS = _NT * _TILE
_PW = 128
_DH = _D // 2
_W = 128
_NWIN = _B // _W


def _gate_body(x_ref, gw_ref, gb_ref,
               scores_ref, pk2_ref, p2_ref, eid_ref):
    x = x_ref[...]
    logits = jnp.dot(x, gw_ref[...], preferred_element_type=jnp.float32)
    logits = logits + gb_ref[...]
    m = jnp.max(logits, axis=1, keepdims=True)
    ex = jnp.exp(logits - m)
    scores = ex / jnp.sum(ex, axis=1, keepdims=True)
    scores_ref[...] = scores

    iota_e = lax.broadcasted_iota(jnp.int32, (_B, _E), 1)
    v1 = jnp.max(scores, axis=1, keepdims=True)
    i1 = jnp.min(jnp.where(scores >= v1, iota_e, _E), axis=1, keepdims=True)
    oh1 = iota_e == i1
    s2 = jnp.where(oh1, -1.0, scores)
    v2 = jnp.max(s2, axis=1, keepdims=True)
    i2 = jnp.min(jnp.where(s2 >= v2, iota_e, _E), axis=1, keepdims=True)
    oh2 = iota_e == i2

    o1 = oh1.astype(jnp.int32)
    o2 = oh2.astype(jnp.int32)

    def excl_cumsum_rows(o):
        s = o
        d = 1
        while d < _B:
            s = s + jnp.concatenate(
                [jnp.zeros((d, _E), jnp.int32), s[: _B - d]], axis=0)
            d *= 2
        return s - o

    r1 = excl_cumsum_rows(o1)
    c1 = jnp.sum(o1, axis=0, keepdims=True)
    r2 = excl_cumsum_rows(o2) + c1
    counts = c1 + jnp.sum(o2, axis=0, keepdims=True)

    cpad = ((counts + (_TILE - 1)) // _TILE) * _TILE
    off = cpad
    d = 1
    while d < _E:
        off = off + jnp.concatenate(
            [jnp.zeros((1, d), jnp.int32), off[:, : _E - d]], axis=1)
        d *= 2
    off = off - cpad
    offb = jnp.broadcast_to(off, (_B, _E))

    pos1 = jnp.sum(o1 * (offb + r1), axis=1, keepdims=True)
    pos2 = jnp.sum(o2 * (offb + r2), axis=1, keepdims=True)

    pk2_ref[0:_B, :] = pos1
    pk2_ref[_B:2 * _B, :] = pos2
    pk2_ref[2 * _B:3 * _B, :] = pos1 + _XS
    pk2_ref[3 * _B:4 * _B, :] = pos2 + _XS

    p2_ref[...] = jnp.concatenate([v1, v2], axis=1)

    tio = lax.broadcasted_iota(jnp.int32, (_NT, _E), 0) * _TILE
    eid = jnp.sum((tio >= jnp.broadcast_to(off, (_NT, _E))).astype(jnp.int32),
                  axis=1, keepdims=True) - 1
    eid_ref[...] = eid


def _gate_call(x, gate_w, gate_b):
    return pl.pallas_call(
        _gate_body,
        out_shape=(
            jax.ShapeDtypeStruct((_B, _E), jnp.float32),
            jax.ShapeDtypeStruct((4 * _B, 1), jnp.int32),
            jax.ShapeDtypeStruct((_B, 2), jnp.float32),
            jax.ShapeDtypeStruct((_NT, 1), jnp.int32),
        ),
    )(x, gate_w, gate_b.reshape(1, _E))


def _sc_mesh():
    return plsc.VectorSubcoreMesh(core_axis_name="c", subcore_axis_name="s")


def _dispatch_call(x, pk2r):
    @functools.partial(
        pl.kernel,
        out_type=jax.ShapeDtypeStruct((2 * _XS, _DH), jnp.float32),
        mesh=_sc_mesh(),
    )
    def k(x_hbm, pos_hbm, xs_hbm):
        def body(x_vmem, i0_vmem, i1_vmem):
            pltpu.sync_copy(x_vmem, xs_hbm.at[i0_vmem.at[0]])
            pltpu.sync_copy(x_vmem, xs_hbm.at[i1_vmem.at[0]])

        pltpu.emit_pipeline(
            body,
            grid=(_NWIN, 2),
            in_specs=[
                pl.BlockSpec((_W, _DH), index_map=lambda i, h: (i, h)),
                pl.BlockSpec((1, _W), index_map=lambda i, h: (h, i)),
                pl.BlockSpec((1, _W), index_map=lambda i, h: (h, _NWIN + i)),
            ],
            out_specs=[],
            core_axis_name=("c", "s"),
            dimension_semantics=(pltpu.PARALLEL, pltpu.PARALLEL),
        )(x_hbm, pos_hbm, pos_hbm)

    return k(x, pk2r)


def _ffn_body(eid_ref, xa_ref, xb_ref, w1a_ref, w1b_ref, b1_ref,
              w2a_ref, w2b_ref, b2a_ref, b2b_ref, ys_ref):
    xa = xa_ref[...].astype(jnp.bfloat16)
    xb = xb_ref[...].astype(jnp.bfloat16)
    h = (jnp.dot(xa, w1a_ref[0], preferred_element_type=jnp.float32)
         + jnp.dot(xb, w1b_ref[0], preferred_element_type=jnp.float32)
         + b1_ref[0])
    h = jnp.maximum(h, 0.0).astype(jnp.bfloat16)
    ys_ref[0] = jnp.dot(h, w2a_ref[0],
                        preferred_element_type=jnp.float32) + b2a_ref[0]
    ys_ref[1] = jnp.dot(h, w2b_ref[0],
                        preferred_element_type=jnp.float32) + b2b_ref[0]


def _ffn_call(xs2, W1, b1, W2, b2, eid):
    grid_spec = pltpu.PrefetchScalarGridSpec(
        num_scalar_prefetch=1,
        grid=(_NT,),
        in_specs=[
            pl.BlockSpec((_TILE, _DH), lambda t, e: (t, 0)),
            pl.BlockSpec((_TILE, _DH), lambda t, e: (_NT + t, 0)),
            pl.BlockSpec((1, _DH, _H), lambda t, e: (e[t], 0, 0)),
            pl.BlockSpec((1, _DH, _H), lambda t, e: (e[t], 1, 0)),
            pl.BlockSpec((1, 1, _H), lambda t, e: (e[t], 0, 0)),
            pl.BlockSpec((1, _H, _DH), lambda t, e: (e[t], 0, 0)),
            pl.BlockSpec((1, _H, _DH), lambda t, e: (e[t], 0, 1)),
            pl.BlockSpec((1, 1, _DH), lambda t, e: (e[t], 0, 0)),
            pl.BlockSpec((1, 1, _DH), lambda t, e: (e[t], 0, 1)),
        ],
        out_specs=pl.BlockSpec((2, _TILE, _DH), lambda t, e: (0, t, 0)),
    )
    call = pl.pallas_call(
        _ffn_body,
        grid_spec=grid_spec,
        out_shape=jax.ShapeDtypeStruct((2, _XS, _DH), jnp.float32),
        compiler_params=pltpu.CompilerParams(
            dimension_semantics=("arbitrary",)),
    )
    W1b = W1.astype(jnp.bfloat16)
    W2b = W2.astype(jnp.bfloat16)
    return call(eid, xs2, xs2, W1b, W1b, b1.reshape(_E, 1, _H),
                W2b, W2b, b2.reshape(_E, 1, _D), b2.reshape(_E, 1, _D))


def _combine_call(ys2, pk2r):
    @functools.partial(
        pl.kernel,
        out_type=jax.ShapeDtypeStruct((4 * _B, _DH), jnp.float32),
        mesh=_sc_mesh(),
    )
    def k(ys_hbm, pos_hbm, yg_hbm):
        def body(i_vmem, out_vmem):
            pltpu.sync_copy(ys_hbm.at[i_vmem.at[0]], out_vmem)

        pltpu.emit_pipeline(
            body,
            grid=(2, (2 * _B) // _W),
            in_specs=[pl.BlockSpec((1, _W), index_map=lambda h, i: (h, i))],
            out_specs=[pl.BlockSpec(
                (_W, _DH),
                index_map=lambda h, i: (h * ((2 * _B) // _W) + i, 0))],
            core_axis_name=("c", "s"),
            dimension_semantics=(pltpu.PARALLEL, pltpu.PARALLEL),
        )(pos_hbm, yg_hbm)

    return k(ys2, pk2r)


_RE = 256


def _padd_body(a0_ref, a1_ref, b0_ref, b1_ref, p_ref, o_ref):
    p0 = p_ref[:, 0:1]
    p1 = p_ref[:, 1:2]
    o_ref[:, 0:_DH] = a0_ref[...] * p0 + a1_ref[...] * p1
    o_ref[:, _DH:_D] = b0_ref[...] * p0 + b1_ref[...] * p1


def _padd_call(yg2, p2):
    nb = _B // _RE
    return pl.pallas_call(
        _padd_body,
        grid=(nb,),
        in_specs=[
            pl.BlockSpec((_RE, _DH), lambda i: (i, 0)),
            pl.BlockSpec((_RE, _DH), lambda i: (_B // _RE + i, 0)),
            pl.BlockSpec((_RE, _DH), lambda i: (2 * _B // _RE + i, 0)),
            pl.BlockSpec((_RE, _DH), lambda i: (3 * _B // _RE + i, 0)),
            pl.BlockSpec((_RE, 2), lambda i: (i, 0)),
        ],
        out_specs=pl.BlockSpec((_RE, _D), lambda i: (i, 0)),
        out_shape=jax.ShapeDtypeStruct((_B, _D), jnp.float32),
    )(yg2, yg2, yg2, yg2, p2)


def kernel(x, gate_w, gate_b, W1, b1, W2, b2):
    scores, pk2, p2, eid_col = _gate_call(x, gate_w, gate_b)
    pk2r = pk2.reshape(2, 2 * _B)
    eid = eid_col.reshape(_NT)
    xs2 = _dispatch_call(x, pk2r)
    ys3 = _ffn_call(xs2, W1, b1, W2, b2, eid)
    yg2 = _combine_call(ys3.reshape(2 * _XS, _DH), pk2r)
    out = _padd_call(yg2, p2)
    return (out, lax.stop_gradient(scores))


# speedup vs baseline: 1.0874x; 1.0874x over previous
import functools

import jax
import jax.numpy as jnp
from jax import lax
from jax.experimental import pallas as pl
from jax.experimental.pallas import tpu as pltpu
from jax.experimental.pallas import tpu_sc as plsc

_B = 2048
_D = 768
_E = 8
_H = 1024
_K = 2
_TILE = 128
_NT = 40
_XS = _NT * _TILE
_PW = 128
_DH = _D // 2
_W = 128
_NWIN = _B // _W


def _gate_body(x_ref, gw_ref, gb_ref,
               scores_ref, pk2_ref, p2_ref, meta_ref):
    x = x_ref[...]
    logits = jnp.dot(x, gw_ref[...], preferred_element_type=jnp.float32)
    logits = logits + gb_ref[...]
    m = jnp.max(logits, axis=1, keepdims=True)
    ex = jnp.exp(logits - m)
    scores = ex / jnp.sum(ex, axis=1, keepdims=True)
    scores_ref[...] = scores

    iota_e = lax.broadcasted_iota(jnp.int32, (_B, _E), 1)
    v1 = jnp.max(scores, axis=1, keepdims=True)
    i1 = jnp.min(jnp.where(scores >= v1, iota_e, _E), axis=1, keepdims=True)
    oh1 = iota_e == i1
    s2 = jnp.where(oh1, -1.0, scores)
    v2 = jnp.max(s2, axis=1, keepdims=True)
    i2 = jnp.min(jnp.where(s2 >= v2, iota_e, _E), axis=1, keepdims=True)
    oh2 = iota_e == i2

    o1 = oh1.astype(jnp.int32)
    o2 = oh2.astype(jnp.int32)

    def excl_cumsum_rows(o):
        s = o
        d = 1
        while d < _B:
            s = s + jnp.concatenate(
                [jnp.zeros((d, _E), jnp.int32), s[: _B - d]], axis=0)
            d *= 2
        return s - o

    r1 = excl_cumsum_rows(o1)
    c1 = jnp.sum(o1, axis=0, keepdims=True)
    r2 = excl_cumsum_rows(o2) + c1
    counts = c1 + jnp.sum(o2, axis=0, keepdims=True)

    cpad = ((counts + (_TILE - 1)) // _TILE) * _TILE
    off = cpad
    d = 1
    while d < _E:
        off = off + jnp.concatenate(
            [jnp.zeros((1, d), jnp.int32), off[:, : _E - d]], axis=1)
        d *= 2
    off = off - cpad
    offb = jnp.broadcast_to(off, (_B, _E))

    pos1 = jnp.sum(o1 * (offb + r1), axis=1, keepdims=True)
    pos2 = jnp.sum(o2 * (offb + r2), axis=1, keepdims=True)

    pk2_ref[0:_B, :] = pos1
    pk2_ref[_B:2 * _B, :] = pos2
    pk2_ref[2 * _B:3 * _B, :] = pos1 + _XS
    pk2_ref[3 * _B:4 * _B, :] = pos2 + _XS

    p2_ref[...] = jnp.concatenate([v1, v2], axis=1)

    tio = lax.broadcasted_iota(jnp.int32, (_NT, _E), 0) * _TILE
    eid = jnp.sum((tio >= jnp.broadcast_to(off, (_NT, _E))).astype(jnp.int32),
                  axis=1, keepdims=True) - 1

    one = jnp.ones((1, 1), jnp.int32)
    zero = jnp.zeros((1, 1), jnp.int32)
    chg = jnp.concatenate(
        [one, (eid[1:] != eid[:-1]).astype(jnp.int32)], axis=0)
    cum = chg
    d = 1
    while d < _NT:
        cum = cum + jnp.concatenate(
            [jnp.zeros((d, 1), jnp.int32), cum[: _NT - d]], axis=0)
        d *= 2
    slot = lax.rem(cum - 1, 2)
    chg1 = jnp.concatenate([chg[1:], zero], axis=0)
    eid1 = jnp.concatenate([eid[1:], zero], axis=0)
    slot1 = jnp.concatenate([slot[1:], zero], axis=0)
    meta_ref[...] = jnp.concatenate(
        [eid, chg, slot, chg1, eid1, slot1], axis=1)


def _gate_call(x, gate_w, gate_b):
    return pl.pallas_call(
        _gate_body,
        out_shape=(
            jax.ShapeDtypeStruct((_B, _E), jnp.float32),
            jax.ShapeDtypeStruct((4 * _B, 1), jnp.int32),
            jax.ShapeDtypeStruct((_B, 2), jnp.float32),
            jax.ShapeDtypeStruct((_NT, 6), jnp.int32),
        ),
    )(x, gate_w, gate_b.reshape(1, _E))


def _sc_mesh():
    return plsc.VectorSubcoreMesh(core_axis_name="c", subcore_axis_name="s")


def _dispatch_call(x, pk2r):
    @functools.partial(
        pl.kernel,
        out_type=jax.ShapeDtypeStruct((2 * _XS, _DH), jnp.float32),
        mesh=_sc_mesh(),
    )
    def k(x_hbm, pos_hbm, xs_hbm):
        def body(x_vmem, i0_vmem, i1_vmem):
            pltpu.sync_copy(x_vmem, xs_hbm.at[i0_vmem.at[0]])
            pltpu.sync_copy(x_vmem, xs_hbm.at[i1_vmem.at[0]])

        pltpu.emit_pipeline(
            body,
            grid=(_NWIN, 2),
            in_specs=[
                pl.BlockSpec((_W, _DH), index_map=lambda i, h: (i, h)),
                pl.BlockSpec((1, _W), index_map=lambda i, h: (h, i)),
                pl.BlockSpec((1, _W), index_map=lambda i, h: (h, _NWIN + i)),
            ],
            out_specs=[],
            core_axis_name=("c", "s"),
            dimension_semantics=(pltpu.PARALLEL, pltpu.PARALLEL),
        )(x_hbm, pos_hbm, pos_hbm)

    return k(x, pk2r)


def _ffn_body(m_ref, xa_ref, xb_ref, b1_ref, b2a_ref, b2b_ref,
              w1_any, w2_any, ys_ref,
              w1f, w2f, w1b, w2b, sem1, sem2):
    t = pl.program_id(0)
    eid = m_ref[t, 0]
    chg = m_ref[t, 1]
    slot = m_ref[t, 2]
    chg1 = m_ref[t, 3]
    eid1 = m_ref[t, 4]
    slot1 = m_ref[t, 5]

    def cp1(e, s):
        return pltpu.make_async_copy(w1_any.at[e], w1f.at[s], sem1.at[s])

    def cp2(e, s):
        return pltpu.make_async_copy(w2_any.at[e], w2f.at[s], sem2.at[s])

    @pl.when(t == 0)
    def _():
        cp1(eid, slot).start()
        cp2(eid, slot).start()

    @pl.when(chg == 1)
    def _():
        cp1(eid, slot).wait()
        cp2(eid, slot).wait()

        @pl.when(slot == 0)
        def _():
            w1b[...] = w1f[0].astype(jnp.bfloat16)
            w2b[...] = w2f[0].astype(jnp.bfloat16)

        @pl.when(slot == 1)
        def _():
            w1b[...] = w1f[1].astype(jnp.bfloat16)
            w2b[...] = w2f[1].astype(jnp.bfloat16)

    @pl.when(chg1 == 1)
    def _():
        cp1(eid1, slot1).start()
        cp2(eid1, slot1).start()

    xa = xa_ref[...].astype(jnp.bfloat16)
    xb = xb_ref[...].astype(jnp.bfloat16)
    h = (jnp.dot(xa, w1b[0:_DH, :], preferred_element_type=jnp.float32)
         + jnp.dot(xb, w1b[_DH:_D, :], preferred_element_type=jnp.float32)
         + b1_ref[0])
    h = jnp.maximum(h, 0.0).astype(jnp.bfloat16)
    ys_ref[0] = jnp.dot(h, w2b[:, 0:_DH],
                        preferred_element_type=jnp.float32) + b2a_ref[0]
    ys_ref[1] = jnp.dot(h, w2b[:, _DH:_D],
                        preferred_element_type=jnp.float32) + b2b_ref[0]


def _ffn_call(xs2, W1, b1, W2, b2, meta):
    grid_spec = pltpu.PrefetchScalarGridSpec(
        num_scalar_prefetch=1,
        grid=(_NT,),
        in_specs=[
            pl.BlockSpec((_TILE, _DH), lambda t, m: (t, 0)),
            pl.BlockSpec((_TILE, _DH), lambda t, m: (_NT + t, 0)),
            pl.BlockSpec((1, 1, _H), lambda t, m: (m[t, 0], 0, 0)),
            pl.BlockSpec((1, 1, _DH), lambda t, m: (m[t, 0], 0, 0)),
            pl.BlockSpec((1, 1, _DH), lambda t, m: (m[t, 0], 0, 1)),
            pl.BlockSpec(memory_space=pl.ANY),
            pl.BlockSpec(memory_space=pl.ANY),
        ],
        out_specs=pl.BlockSpec((2, _TILE, _DH), lambda t, m: (0, t, 0)),
        scratch_shapes=[
            pltpu.VMEM((2, _D, _H), jnp.float32),
            pltpu.VMEM((2, _H, _D), jnp.float32),
            pltpu.VMEM((_D, _H), jnp.bfloat16),
            pltpu.VMEM((_H, _D), jnp.bfloat16),
            pltpu.SemaphoreType.DMA((2,)),
            pltpu.SemaphoreType.DMA((2,)),
        ],
    )
    return pl.pallas_call(
        _ffn_body,
        grid_spec=grid_spec,
        out_shape=jax.ShapeDtypeStruct((2, _XS, _DH), jnp.float32),
        compiler_params=pltpu.CompilerParams(
            dimension_semantics=("arbitrary",)),
    )(meta, xs2, xs2, b1.reshape(_E, 1, _H),
      b2.reshape(_E, 1, _D), b2.reshape(_E, 1, _D), W1, W2)


def _combine_call(ys2, pk2r):
    @functools.partial(
        pl.kernel,
        out_type=jax.ShapeDtypeStruct((4 * _B, _DH), jnp.float32),
        mesh=_sc_mesh(),
    )
    def k(ys_hbm, pos_hbm, yg_hbm):
        def body(i_vmem, out_vmem):
            pltpu.sync_copy(ys_hbm.at[i_vmem.at[0]], out_vmem)

        pltpu.emit_pipeline(
            body,
            grid=(2, (2 * _B) // _W),
            in_specs=[pl.BlockSpec((1, _W), index_map=lambda h, i: (h, i))],
            out_specs=[pl.BlockSpec(
                (_W, _DH),
                index_map=lambda h, i: (h * ((2 * _B) // _W) + i, 0))],
            core_axis_name=("c", "s"),
            dimension_semantics=(pltpu.PARALLEL, pltpu.PARALLEL),
        )(pos_hbm, yg_hbm)

    return k(ys2, pk2r)


_RE = 256


def _padd_body(a0_ref, a1_ref, b0_ref, b1_ref, p_ref, o_ref):
    p0 = p_ref[:, 0:1]
    p1 = p_ref[:, 1:2]
    o_ref[:, 0:_DH] = a0_ref[...] * p0 + a1_ref[...] * p1
    o_ref[:, _DH:_D] = b0_ref[...] * p0 + b1_ref[...] * p1


def _padd_call(yg2, p2):
    nb = _B // _RE
    return pl.pallas_call(
        _padd_body,
        grid=(nb,),
        in_specs=[
            pl.BlockSpec((_RE, _DH), lambda i: (i, 0)),
            pl.BlockSpec((_RE, _DH), lambda i: (_B // _RE + i, 0)),
            pl.BlockSpec((_RE, _DH), lambda i: (2 * _B // _RE + i, 0)),
            pl.BlockSpec((_RE, _DH), lambda i: (3 * _B // _RE + i, 0)),
            pl.BlockSpec((_RE, 2), lambda i: (i, 0)),
        ],
        out_specs=pl.BlockSpec((_RE, _D), lambda i: (i, 0)),
        out_shape=jax.ShapeDtypeStruct((_B, _D), jnp.float32),
    )(yg2, yg2, yg2, yg2, p2)


def kernel(x, gate_w, gate_b, W1, b1, W2, b2):
    scores, pk2, p2, meta = _gate_call(x, gate_w, gate_b)
    pk2r = pk2.reshape(2, 2 * _B)
    xs2 = _dispatch_call(x, pk2r)
    ys3 = _ffn_call(xs2, W1, b1, W2, b2, meta)
    yg2 = _combine_call(ys3.reshape(2 * _XS, _DH), pk2r)
    out = _padd_call(yg2, p2)
    return (out, lax.stop_gradient(scores))


# speedup vs baseline: 1.1225x; 1.0324x over previous
import functools

import jax
import jax.numpy as jnp
from jax import lax
from jax.experimental import pallas as pl
from jax.experimental.pallas import tpu as pltpu
from jax.experimental.pallas import tpu_sc as plsc

_B = 2048
_D = 768
_E = 8
_H = 1024
_K = 2
_TILE = 128
_NT = 40
_XS = _NT * _TILE
_PW = 128
_DH = _D // 2
_W = 128
_NWIN = _B // _W


def _gate_body(x_ref, gw_ref, gb_ref,
               scores_ref, pk2_ref, p2_ref, meta_ref):
    x = x_ref[...]
    logits = jnp.dot(x, gw_ref[...], preferred_element_type=jnp.float32)
    logits = logits + gb_ref[...]
    m = jnp.max(logits, axis=1, keepdims=True)
    ex = jnp.exp(logits - m)
    scores = ex / jnp.sum(ex, axis=1, keepdims=True)
    scores_ref[...] = scores

    iota_e = lax.broadcasted_iota(jnp.int32, (_B, _E), 1)
    v1 = jnp.max(scores, axis=1, keepdims=True)
    i1 = jnp.min(jnp.where(scores >= v1, iota_e, _E), axis=1, keepdims=True)
    oh1 = iota_e == i1
    s2 = jnp.where(oh1, -1.0, scores)
    v2 = jnp.max(s2, axis=1, keepdims=True)
    i2 = jnp.min(jnp.where(s2 >= v2, iota_e, _E), axis=1, keepdims=True)
    oh2 = iota_e == i2

    o1 = oh1.astype(jnp.int32)
    o2 = oh2.astype(jnp.int32)

    def excl_cumsum_rows(o):
        s = o
        d = 1
        while d < _B:
            s = s + jnp.concatenate(
                [jnp.zeros((d, _E), jnp.int32), s[: _B - d]], axis=0)
            d *= 2
        return s - o

    r1 = excl_cumsum_rows(o1)
    c1 = jnp.sum(o1, axis=0, keepdims=True)
    r2 = excl_cumsum_rows(o2) + c1
    counts = c1 + jnp.sum(o2, axis=0, keepdims=True)

    cpad = ((counts + (_TILE - 1)) // _TILE) * _TILE
    off = cpad
    d = 1
    while d < _E:
        off = off + jnp.concatenate(
            [jnp.zeros((1, d), jnp.int32), off[:, : _E - d]], axis=1)
        d *= 2
    off = off - cpad
    offb = jnp.broadcast_to(off, (_B, _E))

    pos1 = jnp.sum(o1 * (offb + r1), axis=1, keepdims=True)
    pos2 = jnp.sum(o2 * (offb + r2), axis=1, keepdims=True)

    pk2_ref[0:_B, :] = pos1
    pk2_ref[_B:2 * _B, :] = pos2
    pk2_ref[2 * _B:3 * _B, :] = pos1 + _XS
    pk2_ref[3 * _B:4 * _B, :] = pos2 + _XS

    p2_ref[...] = jnp.concatenate([v1, v2], axis=1)

    tio = lax.broadcasted_iota(jnp.int32, (_NT, _E), 0) * _TILE
    eid = jnp.sum((tio >= jnp.broadcast_to(off, (_NT, _E))).astype(jnp.int32),
                  axis=1, keepdims=True) - 1

    one = jnp.ones((1, 1), jnp.int32)
    zero = jnp.zeros((1, 1), jnp.int32)
    chg = jnp.concatenate(
        [one, (eid[1:] != eid[:-1]).astype(jnp.int32)], axis=0)
    cum = chg
    d = 1
    while d < _NT:
        cum = cum + jnp.concatenate(
            [jnp.zeros((d, 1), jnp.int32), cum[: _NT - d]], axis=0)
        d *= 2
    slot = lax.rem(cum - 1, 2)
    chg1 = jnp.concatenate([chg[1:], zero], axis=0)
    eid1 = jnp.concatenate([eid[1:], zero], axis=0)
    slot1 = jnp.concatenate([slot[1:], zero], axis=0)
    meta_ref[...] = jnp.concatenate(
        [eid, chg, slot, chg1, eid1, slot1], axis=1)


def _gate_call(x, gate_w, gate_b):
    return pl.pallas_call(
        _gate_body,
        out_shape=(
            jax.ShapeDtypeStruct((_B, _E), jnp.float32),
            jax.ShapeDtypeStruct((4 * _B, 1), jnp.int32),
            jax.ShapeDtypeStruct((_B, 2), jnp.float32),
            jax.ShapeDtypeStruct((_NT, 6), jnp.int32),
        ),
    )(x, gate_w, gate_b.reshape(1, _E))


def _sc_mesh():
    return plsc.VectorSubcoreMesh(core_axis_name="c", subcore_axis_name="s")


def _dispatch_call(x, pk2r):
    @functools.partial(
        pl.kernel,
        out_type=jax.ShapeDtypeStruct((2 * _XS, _DH), jnp.float32),
        mesh=_sc_mesh(),
    )
    def k(x_hbm, pos_hbm, xs_hbm):
        def body(x_vmem, i0_vmem, i1_vmem):
            pltpu.sync_copy(x_vmem, xs_hbm.at[i0_vmem.at[0]])
            pltpu.sync_copy(x_vmem, xs_hbm.at[i1_vmem.at[0]])

        pltpu.emit_pipeline(
            body,
            grid=(_NWIN, 2),
            in_specs=[
                pl.BlockSpec((_W, _DH), index_map=lambda i, h: (i, h)),
                pl.BlockSpec((1, _W), index_map=lambda i, h: (h, i)),
                pl.BlockSpec((1, _W), index_map=lambda i, h: (h, _NWIN + i)),
            ],
            out_specs=[],
            core_axis_name=("c", "s"),
            dimension_semantics=(pltpu.PARALLEL, pltpu.PARALLEL),
        )(x_hbm, pos_hbm, pos_hbm)

    return k(x, pk2r)


def _ffn_body(m_ref, xa_ref, xb_ref, b1_ref, b2a_ref, b2b_ref,
              w1_any, w2_any, ys_ref,
              w1f, w2f, w1b, w2b, sem1, sem2):
    t = pl.program_id(0)
    eid = m_ref[t, 0]
    chg = m_ref[t, 1]
    slot = m_ref[t, 2]
    chg1 = m_ref[t, 3]
    eid1 = m_ref[t, 4]
    slot1 = m_ref[t, 5]

    def cp1(e, s):
        return pltpu.make_async_copy(w1_any.at[e], w1f.at[s], sem1.at[s])

    def cp2(e, s):
        return pltpu.make_async_copy(w2_any.at[e], w2f.at[s], sem2.at[s])

    @pl.when(t == 0)
    def _():
        cp1(eid, slot).start()
        cp2(eid, slot).start()

    @pl.when(chg == 1)
    def _():
        cp1(eid, slot).wait()
        cp2(eid, slot).wait()

        @pl.when(slot == 0)
        def _():
            w1b[...] = w1f[0].astype(jnp.bfloat16)
            w2b[...] = w2f[0].astype(jnp.bfloat16)

        @pl.when(slot == 1)
        def _():
            w1b[...] = w1f[1].astype(jnp.bfloat16)
            w2b[...] = w2f[1].astype(jnp.bfloat16)

    @pl.when(chg1 == 1)
    def _():
        cp1(eid1, slot1).start()
        cp2(eid1, slot1).start()

    xf = jnp.concatenate([xa_ref[...], xb_ref[...]], axis=1)
    h = jnp.dot(xf.astype(jnp.bfloat16), w1b[...],
                preferred_element_type=jnp.float32) + b1_ref[0]
    h = jnp.maximum(h, 0.0).astype(jnp.bfloat16)
    y = jnp.dot(h, w2b[...], preferred_element_type=jnp.float32)
    ys_ref[0] = y[:, 0:_DH] + b2a_ref[0]
    ys_ref[1] = y[:, _DH:_D] + b2b_ref[0]


def _ffn_call(xs2, W1, b1, W2, b2, meta):
    grid_spec = pltpu.PrefetchScalarGridSpec(
        num_scalar_prefetch=1,
        grid=(_NT,),
        in_specs=[
            pl.BlockSpec((_TILE, _DH), lambda t, m: (t, 0)),
            pl.BlockSpec((_TILE, _DH), lambda t, m: (_NT + t, 0)),
            pl.BlockSpec((1, 1, _H), lambda t, m: (m[t, 0], 0, 0)),
            pl.BlockSpec((1, 1, _DH), lambda t, m: (m[t, 0], 0, 0)),
            pl.BlockSpec((1, 1, _DH), lambda t, m: (m[t, 0], 0, 1)),
            pl.BlockSpec(memory_space=pl.ANY),
            pl.BlockSpec(memory_space=pl.ANY),
        ],
        out_specs=pl.BlockSpec((2, _TILE, _DH), lambda t, m: (0, t, 0)),
        scratch_shapes=[
            pltpu.VMEM((2, _D, _H), jnp.float32),
            pltpu.VMEM((2, _H, _D), jnp.float32),
            pltpu.VMEM((_D, _H), jnp.bfloat16),
            pltpu.VMEM((_H, _D), jnp.bfloat16),
            pltpu.SemaphoreType.DMA((2,)),
            pltpu.SemaphoreType.DMA((2,)),
        ],
    )
    return pl.pallas_call(
        _ffn_body,
        grid_spec=grid_spec,
        out_shape=jax.ShapeDtypeStruct((2, _XS, _DH), jnp.float32),
        compiler_params=pltpu.CompilerParams(
            dimension_semantics=("arbitrary",)),
    )(meta, xs2, xs2, b1.reshape(_E, 1, _H),
      b2.reshape(_E, 1, _D), b2.reshape(_E, 1, _D), W1, W2)


def _combine_call(ys2, pk2r):
    @functools.partial(
        pl.kernel,
        out_type=jax.ShapeDtypeStruct((4 * _B, _DH), jnp.float32),
        mesh=_sc_mesh(),
    )
    def k(ys_hbm, pos_hbm, yg_hbm):
        def body(i_vmem, out_vmem):
            pltpu.sync_copy(ys_hbm.at[i_vmem.at[0]], out_vmem)

        pltpu.emit_pipeline(
            body,
            grid=(2, (2 * _B) // _W),
            in_specs=[pl.BlockSpec((1, _W), index_map=lambda h, i: (h, i))],
            out_specs=[pl.BlockSpec(
                (_W, _DH),
                index_map=lambda h, i: (h * ((2 * _B) // _W) + i, 0))],
            core_axis_name=("c", "s"),
            dimension_semantics=(pltpu.PARALLEL, pltpu.PARALLEL),
        )(pos_hbm, yg_hbm)

    return k(ys2, pk2r)


_RE = 512


def _padd_body(a0_ref, a1_ref, b0_ref, b1_ref, p_ref, o_ref):
    p0 = p_ref[:, 0:1]
    p1 = p_ref[:, 1:2]
    o_ref[:, 0:_DH] = a0_ref[...] * p0 + a1_ref[...] * p1
    o_ref[:, _DH:_D] = b0_ref[...] * p0 + b1_ref[...] * p1


def _padd_call(yg2, p2):
    nb = _B // _RE
    return pl.pallas_call(
        _padd_body,
        grid=(nb,),
        in_specs=[
            pl.BlockSpec((_RE, _DH), lambda i: (i, 0)),
            pl.BlockSpec((_RE, _DH), lambda i: (_B // _RE + i, 0)),
            pl.BlockSpec((_RE, _DH), lambda i: (2 * _B // _RE + i, 0)),
            pl.BlockSpec((_RE, _DH), lambda i: (3 * _B // _RE + i, 0)),
            pl.BlockSpec((_RE, 2), lambda i: (i, 0)),
        ],
        out_specs=pl.BlockSpec((_RE, _D), lambda i: (i, 0)),
        out_shape=jax.ShapeDtypeStruct((_B, _D), jnp.float32),
    )(yg2, yg2, yg2, yg2, p2)


def kernel(x, gate_w, gate_b, W1, b1, W2, b2):
    scores, pk2, p2, meta = _gate_call(x, gate_w, gate_b)
    pk2r = pk2.reshape(2, 2 * _B)
    xs2 = _dispatch_call(x, pk2r)
    ys3 = _ffn_call(xs2, W1, b1, W2, b2, meta)
    yg2 = _combine_call(ys3.reshape(2 * _XS, _DH), pk2r)
    out = _padd_call(yg2, p2)
    return (out, lax.stop_gradient(scores))


# speedup vs baseline: 1.1234x; 1.0008x over previous
import functools

import jax
import jax.numpy as jnp
from jax import lax
from jax.experimental import pallas as pl
from jax.experimental.pallas import tpu as pltpu
from jax.experimental.pallas import tpu_sc as plsc

_B = 2048
_D = 768
_E = 8
_H = 1024
_K = 2
_TILE = 128
_NT = 40
_XS = _NT * _TILE
_PW = 128
_DH = _D // 2
_W = 128
_NWIN = _B // _W


def _gate_body(x_ref, gw_ref, gb_ref,
               scores_ref, pk2_ref, p2_ref, meta_ref):
    x = x_ref[...]
    logits = jnp.dot(x, gw_ref[...], preferred_element_type=jnp.float32)
    logits = logits + gb_ref[...]
    m = jnp.max(logits, axis=1, keepdims=True)
    ex = jnp.exp(logits - m)
    scores = ex / jnp.sum(ex, axis=1, keepdims=True)
    scores_ref[...] = scores

    iota_e = lax.broadcasted_iota(jnp.int32, (_B, _E), 1)
    v1 = jnp.max(scores, axis=1, keepdims=True)
    i1 = jnp.min(jnp.where(scores >= v1, iota_e, _E), axis=1, keepdims=True)
    oh1 = iota_e == i1
    s2 = jnp.where(oh1, -1.0, scores)
    v2 = jnp.max(s2, axis=1, keepdims=True)
    i2 = jnp.min(jnp.where(s2 >= v2, iota_e, _E), axis=1, keepdims=True)
    oh2 = iota_e == i2

    o1 = oh1.astype(jnp.int32)
    o2 = oh2.astype(jnp.int32)

    def excl_cumsum_rows(o):
        s = o
        d = 1
        while d < _B:
            s = s + jnp.concatenate(
                [jnp.zeros((d, _E), jnp.int32), s[: _B - d]], axis=0)
            d *= 2
        return s - o

    r1 = excl_cumsum_rows(o1)
    c1 = jnp.sum(o1, axis=0, keepdims=True)
    r2 = excl_cumsum_rows(o2) + c1
    counts = c1 + jnp.sum(o2, axis=0, keepdims=True)

    cpad = ((counts + (_TILE - 1)) // _TILE) * _TILE
    off = cpad
    d = 1
    while d < _E:
        off = off + jnp.concatenate(
            [jnp.zeros((1, d), jnp.int32), off[:, : _E - d]], axis=1)
        d *= 2
    off = off - cpad
    offb = jnp.broadcast_to(off, (_B, _E))

    pos1 = jnp.sum(o1 * (offb + r1), axis=1, keepdims=True)
    pos2 = jnp.sum(o2 * (offb + r2), axis=1, keepdims=True)

    pk2_ref[0:_B, :] = pos1
    pk2_ref[_B:2 * _B, :] = pos2
    pk2_ref[2 * _B:3 * _B, :] = pos1 + _XS
    pk2_ref[3 * _B:4 * _B, :] = pos2 + _XS

    p2_ref[...] = jnp.concatenate([v1, v2], axis=1)

    tio = lax.broadcasted_iota(jnp.int32, (_NT, _E), 0) * _TILE
    eid = jnp.sum((tio >= jnp.broadcast_to(off, (_NT, _E))).astype(jnp.int32),
                  axis=1, keepdims=True) - 1

    one = jnp.ones((1, 1), jnp.int32)
    zero = jnp.zeros((1, 1), jnp.int32)
    chg = jnp.concatenate(
        [one, (eid[1:] != eid[:-1]).astype(jnp.int32)], axis=0)
    cum = chg
    d = 1
    while d < _NT:
        cum = cum + jnp.concatenate(
            [jnp.zeros((d, 1), jnp.int32), cum[: _NT - d]], axis=0)
        d *= 2
    slot = lax.rem(cum - 1, 2)
    chg1 = jnp.concatenate([chg[1:], zero], axis=0)
    eid1 = jnp.concatenate([eid[1:], zero], axis=0)
    slot1 = jnp.concatenate([slot[1:], zero], axis=0)
    meta_ref[...] = jnp.concatenate(
        [eid, chg, slot, chg1, eid1, slot1], axis=1)


def _gate_call(x, gate_w, gate_b):
    return pl.pallas_call(
        _gate_body,
        out_shape=(
            jax.ShapeDtypeStruct((_B, _E), jnp.float32),
            jax.ShapeDtypeStruct((4 * _B, 1), jnp.int32),
            jax.ShapeDtypeStruct((_B, 2), jnp.float32),
            jax.ShapeDtypeStruct((_NT, 6), jnp.int32),
        ),
    )(x, gate_w, gate_b.reshape(1, _E))


def _sc_mesh():
    return plsc.VectorSubcoreMesh(core_axis_name="c", subcore_axis_name="s")


def _dispatch_call(x, pk2r):
    @functools.partial(
        pl.kernel,
        out_type=jax.ShapeDtypeStruct((2 * _XS, _DH), jnp.float32),
        mesh=_sc_mesh(),
    )
    def k(x_hbm, pos_hbm, xs_hbm):
        def body(x_vmem, i0_vmem, i1_vmem):
            pltpu.sync_copy(x_vmem, xs_hbm.at[i0_vmem.at[0]])
            pltpu.sync_copy(x_vmem, xs_hbm.at[i1_vmem.at[0]])

        pltpu.emit_pipeline(
            body,
            grid=(_NWIN, 2),
            in_specs=[
                pl.BlockSpec((_W, _DH), index_map=lambda i, h: (i, h)),
                pl.BlockSpec((1, _W), index_map=lambda i, h: (h, i)),
                pl.BlockSpec((1, _W), index_map=lambda i, h: (h, _NWIN + i)),
            ],
            out_specs=[],
            core_axis_name=("c", "s"),
            dimension_semantics=(pltpu.PARALLEL, pltpu.PARALLEL),
        )(x_hbm, pos_hbm, pos_hbm)

    return k(x, pk2r)


def _ffn_body(m_ref, xa_ref, xb_ref, b1_ref, b2_ref,
              w1_any, w2_any, ys_ref,
              w1f, w2f, w1b, w2b, sem1, sem2):
    t = pl.program_id(0)
    eid = m_ref[t, 0]
    chg = m_ref[t, 1]
    slot = m_ref[t, 2]
    chg1 = m_ref[t, 3]
    eid1 = m_ref[t, 4]
    slot1 = m_ref[t, 5]

    def cp1(e, s):
        return pltpu.make_async_copy(w1_any.at[e], w1f.at[s], sem1.at[s])

    def cp2(e, s):
        return pltpu.make_async_copy(w2_any.at[e], w2f.at[s], sem2.at[s])

    @pl.when(t == 0)
    def _():
        cp1(eid, slot).start()
        cp2(eid, slot).start()

    @pl.when(chg == 1)
    def _():
        cp1(eid, slot).wait()
        cp2(eid, slot).wait()

        @pl.when(slot == 0)
        def _():
            w1b[...] = w1f[0].astype(jnp.bfloat16)
            w2b[...] = w2f[0].astype(jnp.bfloat16)

        @pl.when(slot == 1)
        def _():
            w1b[...] = w1f[1].astype(jnp.bfloat16)
            w2b[...] = w2f[1].astype(jnp.bfloat16)

    @pl.when(chg1 == 1)
    def _():
        cp1(eid1, slot1).start()
        cp2(eid1, slot1).start()

    xf = jnp.concatenate([xa_ref[...], xb_ref[...]], axis=1)
    h = jnp.dot(xf.astype(jnp.bfloat16), w1b[...],
                preferred_element_type=jnp.float32) + b1_ref[eid]
    h = jnp.maximum(h, 0.0).astype(jnp.bfloat16)
    y = jnp.dot(h, w2b[...], preferred_element_type=jnp.float32)
    b2e = b2_ref[eid]
    ys_ref[0] = y[:, 0:_DH] + b2e[:, 0:_DH]
    ys_ref[1] = y[:, _DH:_D] + b2e[:, _DH:_D]


def _ffn_call(xs2, W1, b1, W2, b2, meta):
    grid_spec = pltpu.PrefetchScalarGridSpec(
        num_scalar_prefetch=1,
        grid=(_NT,),
        in_specs=[
            pl.BlockSpec((_TILE, _DH), lambda t, m: (t, 0)),
            pl.BlockSpec((_TILE, _DH), lambda t, m: (_NT + t, 0)),
            pl.BlockSpec((_E, 1, _H), lambda t, m: (0, 0, 0)),
            pl.BlockSpec((_E, 1, _D), lambda t, m: (0, 0, 0)),
            pl.BlockSpec(memory_space=pl.ANY),
            pl.BlockSpec(memory_space=pl.ANY),
        ],
        out_specs=pl.BlockSpec((2, _TILE, _DH), lambda t, m: (0, t, 0)),
        scratch_shapes=[
            pltpu.VMEM((2, _D, _H), jnp.float32),
            pltpu.VMEM((2, _H, _D), jnp.float32),
            pltpu.VMEM((_D, _H), jnp.bfloat16),
            pltpu.VMEM((_H, _D), jnp.bfloat16),
            pltpu.SemaphoreType.DMA((2,)),
            pltpu.SemaphoreType.DMA((2,)),
        ],
    )
    return pl.pallas_call(
        _ffn_body,
        grid_spec=grid_spec,
        out_shape=jax.ShapeDtypeStruct((2, _XS, _DH), jnp.float32),
        compiler_params=pltpu.CompilerParams(
            dimension_semantics=("arbitrary",)),
    )(meta, xs2, xs2, b1.reshape(_E, 1, _H), b2.reshape(_E, 1, _D), W1, W2)


def _combine_call(ys2, pk2r):
    @functools.partial(
        pl.kernel,
        out_type=jax.ShapeDtypeStruct((4 * _B, _DH), jnp.float32),
        mesh=_sc_mesh(),
    )
    def k(ys_hbm, pos_hbm, yg_hbm):
        def body(i_vmem, out_vmem):
            pltpu.sync_copy(ys_hbm.at[i_vmem.at[0]], out_vmem)

        pltpu.emit_pipeline(
            body,
            grid=(2, (2 * _B) // _W),
            in_specs=[pl.BlockSpec((1, _W), index_map=lambda h, i: (h, i))],
            out_specs=[pl.BlockSpec(
                (_W, _DH),
                index_map=lambda h, i: (h * ((2 * _B) // _W) + i, 0))],
            core_axis_name=("c", "s"),
            dimension_semantics=(pltpu.PARALLEL, pltpu.PARALLEL),
        )(pos_hbm, yg_hbm)

    return k(ys2, pk2r)


_RE = 512


def _padd_body(a0_ref, a1_ref, b0_ref, b1_ref, p_ref, o_ref):
    p0 = p_ref[:, 0:1]
    p1 = p_ref[:, 1:2]
    o_ref[:, 0:_DH] = a0_ref[...] * p0 + a1_ref[...] * p1
    o_ref[:, _DH:_D] = b0_ref[...] * p0 + b1_ref[...] * p1


def _padd_call(yg2, p2):
    nb = _B // _RE
    return pl.pallas_call(
        _padd_body,
        grid=(nb,),
        in_specs=[
            pl.BlockSpec((_RE, _DH), lambda i: (i, 0)),
            pl.BlockSpec((_RE, _DH), lambda i: (_B // _RE + i, 0)),
            pl.BlockSpec((_RE, _DH), lambda i: (2 * _B // _RE + i, 0)),
            pl.BlockSpec((_RE, _DH), lambda i: (3 * _B // _RE + i, 0)),
            pl.BlockSpec((_RE, 2), lambda i: (i, 0)),
        ],
        out_specs=pl.BlockSpec((_RE, _D), lambda i: (i, 0)),
        out_shape=jax.ShapeDtypeStruct((_B, _D), jnp.float32),
    )(yg2, yg2, yg2, yg2, p2)


def kernel(x, gate_w, gate_b, W1, b1, W2, b2):
    scores, pk2, p2, meta = _gate_call(x, gate_w, gate_b)
    pk2r = pk2.reshape(2, 2 * _B)
    xs2 = _dispatch_call(x, pk2r)
    ys3 = _ffn_call(xs2, W1, b1, W2, b2, meta)
    yg2 = _combine_call(ys3.reshape(2 * _XS, _DH), pk2r)
    out = _padd_call(yg2, p2)
    return (out, lax.stop_gradient(scores))
